# baseline (speedup 1.0000x reference)
import functools

import jax
import jax.numpy as jnp
from jax import lax
from jax.experimental import pallas as pl
from jax.experimental.pallas import tpu as pltpu
from jax.experimental.pallas import tpu_sc as plsc

N1, N2 = 25, 10
DIN = 128
H0, H1 = 256, 128
P = 1 + N1 + N1 * N2
BB = 64
NW = 32
SCL = 16


def _act(x):
    return jnp.where(x >= 0, x, 0.01 * x)


def _dot(a, b):
    return jnp.dot(a, b, preferred_element_type=jnp.float32)


def _sc_neighbor_means(user_feat, item_feat):
    b = user_feat.shape[0]
    roots_per_w = b // NW
    mesh = plsc.VectorSubcoreMesh(core_axis_name="c", subcore_axis_name="s")

    @functools.partial(
        pl.kernel,
        mesh=mesh,
        out_type=[jax.ShapeDtypeStruct((N1, b, DIN), jnp.float32),
                  jax.ShapeDtypeStruct((N1, b, DIN), jnp.float32)],
        scratch_types=[pltpu.VMEM((N1 * N2 + 2, DIN), jnp.float32),
                       pltpu.VMEM((N1 * N2 + 2, DIN), jnp.float32),
                       pltpu.VMEM((N1, DIN), jnp.float32),
                       pltpu.SemaphoreType.DMA,
                       pltpu.SemaphoreType.DMA],
    )
    def sc_agg(u_hbm, i_hbm, nsu_hbm, nsi_hbm, buf0, buf1, ob, sem0, sem1):
        wid = lax.axis_index("s") * 2 + lax.axis_index("c")
        base = wid * roots_per_w

        def process(src, dst):
            def cp_in(r, buf, sem):
                return pltpu.make_async_copy(
                    src.at[r, pl.ds(24, N1 * N2 + 2), :], buf, sem)

            def compute_store(buf, r):
                def gbody(g, _):
                    row = 2 + g * N2
                    for v in range(DIN // SCL):
                        sl = pl.ds(v * SCL, SCL)
                        acc = buf[row, sl]
                        for rr in range(1, N2):
                            acc = acc + buf[row + rr, sl]
                        ob[g, sl] = acc * (1.0 / N2)
                    return 0
                lax.fori_loop(0, N1, gbody, 0)
                pltpu.sync_copy(ob, dst.at[:, r, :])

            cp_in(base, buf0, sem0).start()

            def body(j2, _):
                ra = base + 2 * j2
                rb = ra + 1
                cp_in(rb, buf1, sem1).start()
                cp_in(ra, buf0, sem0).wait()
                compute_store(buf0, ra)

                @pl.when(j2 < roots_per_w // 2 - 1)
                def _():
                    cp_in(ra + 2, buf0, sem0).start()

                cp_in(rb, buf1, sem1).wait()
                compute_store(buf1, rb)

                @pl.when(j2 < roots_per_w // 2 - 1)
                def _():
                    cp_in(rb + 2, buf1, sem1).start()

                return 0
            lax.fori_loop(0, roots_per_w // 2, body, 0)

        process(u_hbm, nsu_hbm)
        process(i_hbm, nsi_hbm)

    return sc_agg(user_feat, item_feat)


def _tower(head_ref, ns_ref, w1_ref, b1_ref, w2_ref, b2_ref):
    h0 = head_ref[:, 0, :]
    h1_chunks = [head_ref[:, 1 + n1, :] for n1 in range(N1)]
    acc0 = h1_chunks[0]
    for n1 in range(1, N1):
        acc0 = acc0 + h1_chunks[n1]
    neigh0 = acc0 * (1.0 / N1)
    x1 = jnp.concatenate(h1_chunks, axis=0)
    ns = ns_ref[...].reshape(N1 * BB, DIN)
    w1 = w1_ref[...]
    w1a, w1b = w1[:DIN], w1[DIN:]
    b1 = b1_ref[...]
    h1n = _act(_dot(x1, w1a) + _dot(ns, w1b) + b1)
    accn = h1n[0:BB]
    for n1 in range(1, N1):
        accn = accn + h1n[n1 * BB:(n1 + 1) * BB]
    neigh = accn * (1.0 / N1)
    h0n = _act(_dot(h0, w1a) + _dot(neigh0, w1b) + b1)
    w2 = w2_ref[...]
    w2a, w2b = w2[:H0], w2[H0:]
    h0f = _act(_dot(h0n, w2a) + _dot(neigh, w2b) + b2_ref[...])
    return _act(h0f)


def _fused_kernel(uh_ref, ih_ref, nsu_ref, nsi_ref,
                  w1u_ref, b1u_ref, w2u_ref, b2u_ref,
                  w1i_ref, b1i_ref, w2i_ref, b2i_ref, wl_ref, bl_ref,
                  out_ref):
    uh = _tower(uh_ref, nsu_ref, w1u_ref, b1u_ref, w2u_ref, b2u_ref)
    ih = _tower(ih_ref, nsi_ref, w1i_ref, b1i_ref, w2i_ref, b2i_ref)
    pred = _dot(uh * ih, wl_ref[...]) + bl_ref[...]
    out_ref[...] = jax.nn.sigmoid(pred)


def kernel(sampling_user_feat, sampling_item_feat, W1_u, b1_u, W2_u, b2_u,
           W1_i, b1_i, W2_i, b2_i, W_lin, b_lin):
    b = sampling_user_feat.shape[0]
    ns_u, ns_i = _sc_neighbor_means(sampling_user_feat, sampling_item_feat)
    grid = (b // BB,)
    head_spec = pl.BlockSpec((BB, 32, DIN), lambda i: (i, 0, 0))
    ns_spec = pl.BlockSpec((N1, BB, DIN), lambda i: (0, i, 0))
    w1_spec = pl.BlockSpec((2 * DIN, H0), lambda i: (0, 0))
    b1_spec = pl.BlockSpec((1, H0), lambda i: (0, 0))
    w2_spec = pl.BlockSpec((2 * H0, H1), lambda i: (0, 0))
    b2_spec = pl.BlockSpec((1, H1), lambda i: (0, 0))
    wl_spec = pl.BlockSpec((H1, 2), lambda i: (0, 0))
    bl_spec = pl.BlockSpec((1, 2), lambda i: (0, 0))
    out = pl.pallas_call(
        _fused_kernel,
        grid=grid,
        in_specs=[head_spec, head_spec, ns_spec, ns_spec,
                  w1_spec, b1_spec, w2_spec, b2_spec,
                  w1_spec, b1_spec, w2_spec, b2_spec,
                  wl_spec, bl_spec],
        out_specs=pl.BlockSpec((BB, 2), lambda i: (i, 0)),
        out_shape=jax.ShapeDtypeStruct((b, 2), jnp.float32),
        compiler_params=pltpu.CompilerParams(
            dimension_semantics=("parallel",)),
    )(sampling_user_feat, sampling_item_feat, ns_u, ns_i,
      W1_u, b1_u.reshape(1, H0), W2_u, b2_u.reshape(1, H1),
      W1_i, b1_i.reshape(1, H0), W2_i, b2_i.reshape(1, H1),
      W_lin, b_lin.reshape(1, 2))
    return out


# speedup vs baseline: 1.0002x; 1.0002x over previous
import functools

import jax
import jax.numpy as jnp
from jax import lax
from jax.experimental import pallas as pl
from jax.experimental.pallas import tpu as pltpu
from jax.experimental.pallas import tpu_sc as plsc

N1, N2 = 25, 10
DIN = 128
H0, H1 = 256, 128
P = 1 + N1 + N1 * N2
BB = 64
NW = 32
SCL = 16


def _act(x):
    return jnp.where(x >= 0, x, 0.01 * x)


def _dot(a, b):
    return jnp.dot(a, b, preferred_element_type=jnp.float32)


def _sc_neighbor_means(user_feat, item_feat):
    b = user_feat.shape[0]
    roots_per_w = b // NW
    mesh = plsc.VectorSubcoreMesh(core_axis_name="c", subcore_axis_name="s")

    @functools.partial(
        pl.kernel,
        mesh=mesh,
        out_type=[jax.ShapeDtypeStruct((N1, b // 8, 8, DIN), jnp.float32),
                  jax.ShapeDtypeStruct((N1, b // 8, 8, DIN), jnp.float32)],
        scratch_types=[pltpu.VMEM((N1 * N2 + 2, DIN), jnp.float32),
                       pltpu.VMEM((N1 * N2 + 2, DIN), jnp.float32),
                       pltpu.VMEM((N1, DIN), jnp.float32),
                       pltpu.SemaphoreType.DMA,
                       pltpu.SemaphoreType.DMA],
    )
    def sc_agg(u_hbm, i_hbm, nsu_hbm, nsi_hbm, buf0, buf1, ob, sem0, sem1):
        wid = lax.axis_index("s") * 2 + lax.axis_index("c")
        base = wid * roots_per_w

        def process(src, dst):
            def cp_in(r, buf, sem):
                return pltpu.make_async_copy(
                    src.at[r, pl.ds(24, N1 * N2 + 2), :], buf, sem)

            def compute_store(buf, r):
                def gbody(g, _):
                    row = 2 + g * N2
                    for v in range(DIN // SCL):
                        sl = pl.ds(v * SCL, SCL)
                        acc = buf[row, sl]
                        for rr in range(1, N2):
                            acc = acc + buf[row + rr, sl]
                        ob[g, sl] = acc * (1.0 / N2)
                    return 0
                lax.fori_loop(0, N1, gbody, 0)
                pltpu.sync_copy(ob, dst.at[:, r // 8, r % 8, :])

            cp_in(base, buf0, sem0).start()

            def body(j2, _):
                ra = base + 2 * j2
                rb = ra + 1
                cp_in(rb, buf1, sem1).start()
                cp_in(ra, buf0, sem0).wait()
                compute_store(buf0, ra)

                @pl.when(j2 < roots_per_w // 2 - 1)
                def _():
                    cp_in(ra + 2, buf0, sem0).start()

                cp_in(rb, buf1, sem1).wait()
                compute_store(buf1, rb)

                @pl.when(j2 < roots_per_w // 2 - 1)
                def _():
                    cp_in(rb + 2, buf1, sem1).start()

                return 0
            lax.fori_loop(0, roots_per_w // 2, body, 0)

        process(u_hbm, nsu_hbm)
        process(i_hbm, nsi_hbm)

    return sc_agg(user_feat, item_feat)


def _tower(head_ref, ns_ref, w1_ref, b1_ref, w2_ref, b2_ref):
    h0 = head_ref[:, 0, :]
    h1_chunks = [head_ref[:, 1 + n1, :] for n1 in range(N1)]
    acc0 = h1_chunks[0]
    for n1 in range(1, N1):
        acc0 = acc0 + h1_chunks[n1]
    neigh0 = acc0 * (1.0 / N1)
    x1 = jnp.concatenate(h1_chunks, axis=0)
    ns = ns_ref[...].reshape(N1 * BB, DIN)
    w1 = w1_ref[...]
    w1a, w1b = w1[:DIN], w1[DIN:]
    b1 = b1_ref[...]
    h1n = _act(_dot(x1, w1a) + _dot(ns, w1b) + b1)
    accn = h1n[0:BB]
    for n1 in range(1, N1):
        accn = accn + h1n[n1 * BB:(n1 + 1) * BB]
    neigh = accn * (1.0 / N1)
    h0n = _act(_dot(h0, w1a) + _dot(neigh0, w1b) + b1)
    w2 = w2_ref[...]
    w2a, w2b = w2[:H0], w2[H0:]
    h0f = _act(_dot(h0n, w2a) + _dot(neigh, w2b) + b2_ref[...])
    return _act(h0f)


def _fused_kernel(uh_ref, ih_ref, nsu_ref, nsi_ref,
                  w1u_ref, b1u_ref, w2u_ref, b2u_ref,
                  w1i_ref, b1i_ref, w2i_ref, b2i_ref, wl_ref, bl_ref,
                  out_ref):
    uh = _tower(uh_ref, nsu_ref, w1u_ref, b1u_ref, w2u_ref, b2u_ref)
    ih = _tower(ih_ref, nsi_ref, w1i_ref, b1i_ref, w2i_ref, b2i_ref)
    pred = _dot(uh * ih, wl_ref[...]) + bl_ref[...]
    out_ref[...] = jax.nn.sigmoid(pred)


def kernel(sampling_user_feat, sampling_item_feat, W1_u, b1_u, W2_u, b2_u,
           W1_i, b1_i, W2_i, b2_i, W_lin, b_lin):
    b = sampling_user_feat.shape[0]
    ns_u, ns_i = _sc_neighbor_means(sampling_user_feat, sampling_item_feat)
    grid = (b // BB,)
    head_spec = pl.BlockSpec((BB, 32, DIN), lambda i: (i, 0, 0))
    ns_spec = pl.BlockSpec((N1, BB // 8, 8, DIN), lambda i: (0, i, 0, 0))
    w1_spec = pl.BlockSpec((2 * DIN, H0), lambda i: (0, 0))
    b1_spec = pl.BlockSpec((1, H0), lambda i: (0, 0))
    w2_spec = pl.BlockSpec((2 * H0, H1), lambda i: (0, 0))
    b2_spec = pl.BlockSpec((1, H1), lambda i: (0, 0))
    wl_spec = pl.BlockSpec((H1, 2), lambda i: (0, 0))
    bl_spec = pl.BlockSpec((1, 2), lambda i: (0, 0))
    out = pl.pallas_call(
        _fused_kernel,
        grid=grid,
        in_specs=[head_spec, head_spec, ns_spec, ns_spec,
                  w1_spec, b1_spec, w2_spec, b2_spec,
                  w1_spec, b1_spec, w2_spec, b2_spec,
                  wl_spec, bl_spec],
        out_specs=pl.BlockSpec((BB, 2), lambda i: (i, 0)),
        out_shape=jax.ShapeDtypeStruct((b, 2), jnp.float32),
        compiler_params=pltpu.CompilerParams(
            dimension_semantics=("parallel",)),
    )(sampling_user_feat, sampling_item_feat, ns_u, ns_i,
      W1_u, b1_u.reshape(1, H0), W2_u, b2_u.reshape(1, H1),
      W1_i, b1_i.reshape(1, H0), W2_i, b2_i.reshape(1, H1),
      W_lin, b_lin.reshape(1, 2))
    return out
